# baseline (speedup 1.0000x reference)
import functools

import jax
import jax.numpy as jnp
from jax import lax
from jax.experimental import pallas as pl
from jax.experimental.pallas import tpu as pltpu
from jax.experimental.pallas import tpu_sc as plsc

_N = 50000
_G = 512
_DIN = 256
_DOUT = 128

_NP = 50176
_NW = 32
_IPW = _G // _NW
_BLK = 128
_NB = (_N + _BLK - 1) // _BLK
_NCOL = _DIN // 16

_TBLK = 512
_NB_TC = 66
_SCB0 = 380
_SCK = 124
_SCB1 = 388


def _sc_segment_sum(x, bpad):
    mesh = plsc.VectorSubcoreMesh(core_axis_name="c", subcore_axis_name="s")

    @functools.partial(
        pl.kernel,
        out_type=jax.ShapeDtypeStruct((_G, _DIN), jnp.float32),
        mesh=mesh,
        scratch_types=[
            pltpu.VMEM((_NP,), jnp.int32),
            pltpu.VMEM((_BLK, _DIN), jnp.float32),
            pltpu.VMEM((_BLK, _DIN), jnp.float32),
            pltpu.VMEM((_IPW, _DIN), jnp.float32),
            pltpu.SemaphoreType.DMA,
            pltpu.SemaphoreType.DMA,
        ],
        compiler_params=pltpu.CompilerParams(needs_layout_passes=False),
    )
    def k(x_hbm, b_hbm, out_hbm, ids_v, buf0, buf1, outbuf, sem0, sem1):
        cid = lax.axis_index("c")
        sid = lax.axis_index("s")
        wid = cid * 16 + sid
        o = wid * _IPW

        pltpu.sync_copy(b_hbm, ids_v)

        lanes = lax.iota(jnp.int32, 16)

        def lower_bound(thresh):
            lo = jnp.zeros((16,), jnp.int32)
            hi = jnp.full((16,), _N, jnp.int32)

            def step(_, c):
                lo, hi = c
                mid = (lo + hi) >> 1
                v = plsc.load_gather(ids_v, [mid])
                less = v < thresh
                return (jnp.where(less, mid + 1, lo),
                        jnp.where(less, hi, mid))

            lo, hi = lax.fori_loop(0, 17, step, (lo, hi))
            return lo

        s0 = lower_bound(o + lanes)
        s1 = lower_bound(o + lanes + 1)

        def extract(vec, l):
            return jnp.sum(jnp.where(lanes == l, vec, 0))

        r0s = [extract(s0, l) for l in range(_IPW)]
        r1s = [extract(s1, l) for l in range(_IPW)]

        zv = jnp.zeros((16,), jnp.float32)
        for l in range(_IPW):
            for i in range(_NCOL):
                outbuf[l, pl.ds(i * 16, 16)] = zv

        def sc_count_below(xb):
            base = jnp.minimum(xb, _SCB0)
            f1 = (base // 12) * 4 + jnp.maximum(base % 12 - 8, 0)
            return f1 + jnp.maximum(xb - _SCB1, 0)

        b_lo = r0s[0] >> 7
        b_hi = (r1s[_IPW - 1] - 1) >> 7
        k0 = sc_count_below(b_lo)
        cnt = sc_count_below(b_hi + 1) - k0

        def block_of(j):
            kk = k0 + j
            return jnp.where(kk < _SCK,
                             (kk >> 2) * 12 + 8 + (kk & 3),
                             _SCB1 + (kk - _SCK))

        bufs = (buf0, buf1)
        sems = (sem0, sem1)

        def cstart_of(j):
            return pl.multiple_of(
                jnp.minimum(block_of(j) * _BLK, _N - _BLK), 8)

        def dma(j, slot):
            return pltpu.make_async_copy(
                x_hbm.at[pl.ds(cstart_of(j), _BLK)], bufs[slot], sems[slot])

        def start(j, slot):
            @pl.when(j < cnt)
            def _():
                dma(j, slot).start()

        def process(j, slot):
            @pl.when(j < cnt)
            def _():
                dma(j, slot).wait()
                start_r = cstart_of(j)
                w_lo = block_of(j) * _BLK
                buf = bufs[slot]
                for l in range(_IPW):
                    a = jnp.maximum(r0s[l], w_lo) - start_r
                    b = jnp.maximum(
                        jnp.minimum(r1s[l], w_lo + _BLK) - start_r, a)

                    @pl.when(b > a)
                    def _():
                        acc = tuple(outbuf[l, pl.ds(i * 16, 16)]
                                    for i in range(_NCOL))

                        def row2(p, acc):
                            r = a + 2 * p
                            acc = tuple(acc[i] + buf[r, pl.ds(i * 16, 16)]
                                        for i in range(_NCOL))
                            return tuple(
                                acc[i] + buf[r + 1, pl.ds(i * 16, 16)]
                                for i in range(_NCOL))

                        acc = lax.fori_loop(0, (b - a) >> 1, row2, acc)
                        odd = ((b - a) & 1) == 1
                        acc = tuple(
                            acc[i] + jnp.where(
                                odd, buf[b - 1, pl.ds(i * 16, 16)], zv)
                            for i in range(_NCOL))
                        for i in range(_NCOL):
                            outbuf[l, pl.ds(i * 16, 16)] = acc[i]

        start(0, 0)
        npairs = (cnt + 1) >> 1

        def pair_body(p, carry):
            c0 = 2 * p
            start(c0 + 1, 1)
            process(c0, 0)
            start(c0 + 2, 0)
            process(c0 + 1, 1)
            return carry

        lax.fori_loop(0, npairs, pair_body, 0)

        pltpu.sync_copy(outbuf, out_hbm.at[pl.ds(o, _IPW)])

    return k(x, bpad)


def _tc_block(i):
    return jnp.where(i < 64, 3 * (i // 2) + i % 2, 95 + (i - 64))


def _tc_onehot(x, b3d, gid):
    def body(ids_ref, x_ref, gid_ref, o_ref):
        i = pl.program_id(0)

        @pl.when(i == 0)
        def _():
            o_ref[...] = jnp.zeros_like(o_ref)

        ids = ids_ref[0, 0, :]
        oh = (ids[None, :] == gid_ref[...]).astype(jnp.bfloat16)
        xb = x_ref[...].astype(jnp.bfloat16)
        o_ref[...] += jnp.dot(oh, xb, preferred_element_type=jnp.float32)

    return pl.pallas_call(
        body,
        grid=(_NB_TC,),
        in_specs=[
            pl.BlockSpec((1, 1, _TBLK), lambda i: (_tc_block(i), 0, 0)),
            pl.BlockSpec((_TBLK, _DIN), lambda i: (_tc_block(i), 0)),
            pl.BlockSpec((_G, _TBLK), lambda i: (0, 0)),
        ],
        out_specs=pl.BlockSpec((_G, _DIN), lambda i: (0, 0)),
        out_shape=jax.ShapeDtypeStruct((_G, _DIN), jnp.float32),
    )(b3d, x, gid)


def _tc_head(emb_sc, emb_tc, W, b2d):
    def body(p_ref, q_ref, w_ref, b_ref, o_ref):
        acc = p_ref[...] + q_ref[...]
        o_ref[...] = jnp.dot(acc, w_ref[...],
                             preferred_element_type=jnp.float32) + b_ref[...]

    return pl.pallas_call(
        body,
        out_shape=jax.ShapeDtypeStruct((_G, _DOUT), jnp.float32),
    )(emb_sc, emb_tc, W, b2d)


def kernel(x, batch, y, W, b):
    bpad = jnp.pad(batch, (0, _NP - _N), constant_values=jnp.int32(1 << 30))
    emb_sc = _sc_segment_sum(x, bpad)
    gid = lax.broadcasted_iota(jnp.int32, (_G, _TBLK), 0)
    emb_tc = _tc_onehot(x, bpad.reshape(_NP // _TBLK, 1, _TBLK), gid)
    pred = _tc_head(emb_sc, emb_tc, W, b.reshape(1, _DOUT))
    return (pred, y)


# speedup vs baseline: 1.0190x; 1.0190x over previous
import functools

import jax
import jax.numpy as jnp
from jax import lax
from jax.experimental import pallas as pl
from jax.experimental.pallas import tpu as pltpu
from jax.experimental.pallas import tpu_sc as plsc

_N = 50000
_G = 512
_DIN = 256
_DOUT = 128

_NP = 50176
_NW = 32
_IPW = _G // _NW
_BLK = 128
_NB = (_N + _BLK - 1) // _BLK
_NCOL = _DIN // 16

_TBLK = 512
_NB_TC = 66
_SCB0 = 380
_SCK = 124
_SCB1 = 388


def _sc_segment_sum(x, bpad):
    mesh = plsc.VectorSubcoreMesh(core_axis_name="c", subcore_axis_name="s")

    @functools.partial(
        pl.kernel,
        out_type=jax.ShapeDtypeStruct((_G, _DIN), jnp.float32),
        mesh=mesh,
        scratch_types=[
            pltpu.VMEM((_NP,), jnp.int32),
            pltpu.VMEM((_BLK, _DIN), jnp.float32),
            pltpu.VMEM((_BLK, _DIN), jnp.float32),
            pltpu.VMEM((_IPW, _DIN), jnp.float32),
            pltpu.SemaphoreType.DMA,
            pltpu.SemaphoreType.DMA,
        ],
        compiler_params=pltpu.CompilerParams(needs_layout_passes=False),
    )
    def k(x_hbm, b_hbm, out_hbm, ids_v, buf0, buf1, outbuf, sem0, sem1):
        cid = lax.axis_index("c")
        sid = lax.axis_index("s")
        wid = cid * 16 + sid
        o = wid * _IPW

        pltpu.sync_copy(b_hbm, ids_v)

        lanes = lax.iota(jnp.int32, 16)

        def lower_bound(thresh):
            lo = jnp.zeros((16,), jnp.int32)
            hi = jnp.full((16,), _N, jnp.int32)

            def step(_, c):
                lo, hi = c
                mid = (lo + hi) >> 1
                v = plsc.load_gather(ids_v, [mid])
                less = v < thresh
                return (jnp.where(less, mid + 1, lo),
                        jnp.where(less, hi, mid))

            lo, hi = lax.fori_loop(0, 17, step, (lo, hi))
            return lo

        s0 = lower_bound(o + lanes)
        s1 = lower_bound(o + lanes + 1)

        def extract(vec, l):
            return jnp.sum(jnp.where(lanes == l, vec, 0))

        r0s = [extract(s0, l) for l in range(_IPW)]
        r1s = [extract(s1, l) for l in range(_IPW)]

        zv = jnp.zeros((16,), jnp.float32)
        for l in range(_IPW):
            for i in range(_NCOL):
                outbuf[l, pl.ds(i * 16, 16)] = zv

        def sc_count_below(xb):
            base = jnp.minimum(xb, _SCB0)
            f1 = (base // 12) * 4 + jnp.maximum(base % 12 - 8, 0)
            return f1 + jnp.maximum(xb - _SCB1, 0)

        b_lo = r0s[0] >> 7
        b_hi = (r1s[_IPW - 1] - 1) >> 7
        k0 = sc_count_below(b_lo)
        cnt = sc_count_below(b_hi + 1) - k0

        def block_of(j):
            kk = k0 + j
            return jnp.where(kk < _SCK,
                             (kk >> 2) * 12 + 8 + (kk & 3),
                             _SCB1 + (kk - _SCK))

        bufs = (buf0, buf1)
        sems = (sem0, sem1)

        def cstart_of(j):
            return pl.multiple_of(
                jnp.minimum(block_of(j) * _BLK, _N - _BLK), 8)

        def dma(j, slot):
            return pltpu.make_async_copy(
                x_hbm.at[pl.ds(cstart_of(j), _BLK)], bufs[slot], sems[slot])

        def start(j, slot):
            @pl.when(j < cnt)
            def _():
                dma(j, slot).start()

        def process(j, slot):
            @pl.when(j < cnt)
            def _():
                dma(j, slot).wait()
                start_r = cstart_of(j)
                w_lo = block_of(j) * _BLK
                buf = bufs[slot]
                for l in range(_IPW):
                    a = jnp.maximum(r0s[l], w_lo) - start_r
                    b = jnp.maximum(
                        jnp.minimum(r1s[l], w_lo + _BLK) - start_r, a)

                    @pl.when(b > a)
                    def _():
                        acc = tuple(outbuf[l, pl.ds(i * 16, 16)]
                                    for i in range(_NCOL))

                        def row2(p, acc):
                            r = a + 2 * p
                            acc = tuple(acc[i] + buf[r, pl.ds(i * 16, 16)]
                                        for i in range(_NCOL))
                            return tuple(
                                acc[i] + buf[r + 1, pl.ds(i * 16, 16)]
                                for i in range(_NCOL))

                        acc = lax.fori_loop(0, (b - a) >> 1, row2, acc)
                        odd = ((b - a) & 1) == 1
                        acc = tuple(
                            acc[i] + jnp.where(
                                odd, buf[b - 1, pl.ds(i * 16, 16)], zv)
                            for i in range(_NCOL))
                        for i in range(_NCOL):
                            outbuf[l, pl.ds(i * 16, 16)] = acc[i]

        start(0, 0)
        npairs = (cnt + 1) >> 1

        def pair_body(p, carry):
            c0 = 2 * p
            start(c0 + 1, 1)
            process(c0, 0)
            start(c0 + 2, 0)
            process(c0 + 1, 1)
            return carry

        lax.fori_loop(0, npairs, pair_body, 0)

        pltpu.sync_copy(outbuf, out_hbm.at[pl.ds(o, _IPW)])

    return k(x, bpad)


def _tc_block(i):
    return jnp.where(i < 64, 3 * (i // 2) + i % 2, 95 + (i - 64))


def _tc_onehot(x, b3d):
    def body(ids_ref, x_ref, o_ref):
        i = pl.program_id(0)

        @pl.when(i == 0)
        def _():
            o_ref[...] = jnp.zeros_like(o_ref)

        ids = ids_ref[0, 0, :]
        gid = lax.broadcasted_iota(jnp.int32, (_G, _TBLK), 0)
        oh = (ids[None, :] == gid).astype(jnp.bfloat16)
        xb = x_ref[...].astype(jnp.bfloat16)
        o_ref[...] += jnp.dot(oh, xb, preferred_element_type=jnp.float32)

    return pl.pallas_call(
        body,
        grid=(_NB_TC,),
        in_specs=[
            pl.BlockSpec((1, 1, _TBLK), lambda i: (_tc_block(i), 0, 0)),
            pl.BlockSpec((_TBLK, _DIN), lambda i: (_tc_block(i), 0)),
        ],
        out_specs=pl.BlockSpec((_G, _DIN), lambda i: (0, 0)),
        out_shape=jax.ShapeDtypeStruct((_G, _DIN), jnp.float32),
    )(b3d, x)


def _tc_head(emb_sc, emb_tc, W, b2d):
    def body(p_ref, q_ref, w_ref, b_ref, o_ref):
        acc = p_ref[...] + q_ref[...]
        o_ref[...] = jnp.dot(acc, w_ref[...],
                             preferred_element_type=jnp.float32) + b_ref[...]

    return pl.pallas_call(
        body,
        out_shape=jax.ShapeDtypeStruct((_G, _DOUT), jnp.float32),
    )(emb_sc, emb_tc, W, b2d)


def kernel(x, batch, y, W, b):
    bpad = jnp.pad(batch, (0, _NP - _N), constant_values=jnp.int32(1 << 30))
    emb_sc = _sc_segment_sum(x, bpad)
    emb_tc = _tc_onehot(x, bpad.reshape(_NP // _TBLK, 1, _TBLK))
    pred = _tc_head(emb_sc, emb_tc, W, b.reshape(1, _DOUT))
    return (pred, y)


# speedup vs baseline: 1.0512x; 1.0316x over previous
import functools

import jax
import jax.numpy as jnp
from jax import lax
from jax.experimental import pallas as pl
from jax.experimental.pallas import tpu as pltpu
from jax.experimental.pallas import tpu_sc as plsc

_N = 50000
_G = 512
_DIN = 256
_DOUT = 128

_NP = 50176
_NS = 3200
_NW = 32
_IPW = _G // _NW
_BLK = 128
_NB = (_N + _BLK - 1) // _BLK
_NCOL = _DIN // 16

_TBLK = 512
_NB_TC = 66
_SCB0 = 380
_SCK = 124
_SCB1 = 388


def _sc_segment_sum(x, bpad, bsmall):
    mesh = plsc.VectorSubcoreMesh(core_axis_name="c", subcore_axis_name="s")

    @functools.partial(
        pl.kernel,
        out_type=jax.ShapeDtypeStruct((_G, _DIN), jnp.float32),
        mesh=mesh,
        scratch_types=[
            pltpu.VMEM((_NS,), jnp.int32),
            pltpu.VMEM((32, 32), jnp.int32),
            pltpu.VMEM((_BLK, _DIN), jnp.float32),
            pltpu.VMEM((_BLK, _DIN), jnp.float32),
            pltpu.VMEM((_BLK, _DIN), jnp.float32),
            pltpu.VMEM((_IPW, _DIN), jnp.float32),
            pltpu.SemaphoreType.DMA,
            pltpu.SemaphoreType.DMA,
            pltpu.SemaphoreType.DMA,
            pltpu.SemaphoreType.DMA,
        ],
        compiler_params=pltpu.CompilerParams(needs_layout_passes=False),
    )
    def k(x_hbm, b_hbm, bs_hbm, out_hbm, bsm_v, winbuf,
          buf0, buf1, buf2, outbuf, sem0, sem1, sem2, wsem):
        cid = lax.axis_index("c")
        sid = lax.axis_index("s")
        wid = cid * 16 + sid
        o = wid * _IPW

        lanes = lax.iota(jnp.int32, 16)

        def extract(vec, l):
            return jnp.sum(jnp.where(lanes == l, vec, 0))

        pltpu.sync_copy(bs_hbm, bsm_v)

        def coarse(thresh):
            lo = jnp.zeros((16,), jnp.int32)
            hi = jnp.full((16,), _NP // 16, jnp.int32)

            def step(_, c):
                lo, hi = c
                mid = (lo + hi) >> 1
                v = plsc.load_gather(bsm_v, [mid])
                less = v < thresh
                return (jnp.where(less, mid + 1, lo),
                        jnp.where(less, hi, mid))

            lo, hi = lax.fori_loop(0, 12, step, (lo, hi))
            return lo

        t0 = o + lanes
        t1 = o + lanes + 1
        w0 = jnp.clip((coarse(t0) - 1) * 16, 0, _NP - 32)
        w1 = jnp.clip((coarse(t1) - 1) * 16, 0, _NP - 32)

        wcopies = []
        for l in range(_IPW):
            for row, wv in ((l, w0), (16 + l, w1)):
                off = pl.multiple_of(extract(wv, l), 16)
                cp = pltpu.make_async_copy(
                    b_hbm.at[pl.ds(off, 32)], winbuf.at[row], wsem)
                cp.start()
                wcopies.append(cp)
        for cp in wcopies:
            cp.wait()

        def refine(rowbase, thresh):
            lo = jnp.zeros((16,), jnp.int32)
            hi = jnp.full((16,), 32, jnp.int32)

            def step(_, c):
                lo, hi = c
                mid = (lo + hi) >> 1
                v = plsc.load_gather(winbuf, [rowbase, mid])
                less = v < thresh
                return (jnp.where(less, mid + 1, lo),
                        jnp.where(less, hi, mid))

            lo, hi = lax.fori_loop(0, 5, step, (lo, hi))
            return lo

        s0 = w0 + refine(lanes, t0)
        s1 = w1 + refine(lanes + 16, t1)

        r0s = [extract(s0, l) for l in range(_IPW)]
        r1s = [extract(s1, l) for l in range(_IPW)]

        zv = jnp.zeros((16,), jnp.float32)
        for l in range(_IPW):
            for i in range(_NCOL):
                outbuf[l, pl.ds(i * 16, 16)] = zv

        def sc_count_below(xb):
            base = jnp.minimum(xb, _SCB0)
            f1 = (base // 12) * 4 + jnp.maximum(base % 12 - 8, 0)
            return f1 + jnp.maximum(xb - _SCB1, 0)

        b_lo = r0s[0] >> 7
        b_hi = (r1s[_IPW - 1] - 1) >> 7
        k0 = sc_count_below(b_lo)
        cnt = sc_count_below(b_hi + 1) - k0

        def block_of(j):
            kk = k0 + j
            return jnp.where(kk < _SCK,
                             (kk >> 2) * 12 + 8 + (kk & 3),
                             _SCB1 + (kk - _SCK))

        bufs = (buf0, buf1, buf2)
        sems = (sem0, sem1, sem2)

        def cstart_of(j):
            return pl.multiple_of(
                jnp.minimum(block_of(j) * _BLK, _N - _BLK), 8)

        def dma(j, slot):
            return pltpu.make_async_copy(
                x_hbm.at[pl.ds(cstart_of(j), _BLK)], bufs[slot], sems[slot])

        def start(j, slot):
            @pl.when(j < cnt)
            def _():
                dma(j, slot).start()

        def process(j, slot):
            @pl.when(j < cnt)
            def _():
                dma(j, slot).wait()
                start_r = cstart_of(j)
                w_lo = block_of(j) * _BLK
                buf = bufs[slot]
                for l in range(_IPW):
                    a = jnp.maximum(r0s[l], w_lo) - start_r
                    b = jnp.maximum(
                        jnp.minimum(r1s[l], w_lo + _BLK) - start_r, a)

                    @pl.when(b > a)
                    def _():
                        acc = tuple(outbuf[l, pl.ds(i * 16, 16)]
                                    for i in range(_NCOL))

                        def row2(p, acc):
                            r = a + 2 * p
                            acc = tuple(acc[i] + buf[r, pl.ds(i * 16, 16)]
                                        for i in range(_NCOL))
                            return tuple(
                                acc[i] + buf[r + 1, pl.ds(i * 16, 16)]
                                for i in range(_NCOL))

                        acc = lax.fori_loop(0, (b - a) >> 1, row2, acc)
                        odd = ((b - a) & 1) == 1
                        acc = tuple(
                            acc[i] + jnp.where(
                                odd, buf[b - 1, pl.ds(i * 16, 16)], zv)
                            for i in range(_NCOL))
                        for i in range(_NCOL):
                            outbuf[l, pl.ds(i * 16, 16)] = acc[i]

        start(0, 0)
        start(1, 1)
        ntrips = (cnt + 2) // 3

        def trip_body(p, carry):
            c0 = 3 * p
            start(c0 + 2, 2)
            process(c0, 0)
            start(c0 + 3, 0)
            process(c0 + 1, 1)
            start(c0 + 4, 1)
            process(c0 + 2, 2)
            return carry

        lax.fori_loop(0, ntrips, trip_body, 0)

        pltpu.sync_copy(outbuf, out_hbm.at[pl.ds(o, _IPW)])

    return k(x, bpad, bsmall)


def _tc_block(i):
    return jnp.where(i < 64, 3 * (i // 2) + i % 2, 95 + (i - 64))


def _tc_onehot(x, b3d):
    def body(ids_ref, x_ref, o_ref):
        i = pl.program_id(0)

        @pl.when(i == 0)
        def _():
            o_ref[...] = jnp.zeros_like(o_ref)

        ids = ids_ref[0, 0, :]
        gid = lax.broadcasted_iota(jnp.int32, (_G, _TBLK), 0)
        oh = (ids[None, :] == gid).astype(jnp.bfloat16)
        xb = x_ref[...].astype(jnp.bfloat16)
        o_ref[...] += jnp.dot(oh, xb, preferred_element_type=jnp.float32)

    return pl.pallas_call(
        body,
        grid=(_NB_TC,),
        in_specs=[
            pl.BlockSpec((1, 1, _TBLK), lambda i: (_tc_block(i), 0, 0)),
            pl.BlockSpec((_TBLK, _DIN), lambda i: (_tc_block(i), 0)),
        ],
        out_specs=pl.BlockSpec((_G, _DIN), lambda i: (0, 0)),
        out_shape=jax.ShapeDtypeStruct((_G, _DIN), jnp.float32),
    )(b3d, x)


def _tc_head(emb_sc, emb_tc, W, b2d):
    def body(p_ref, q_ref, w_ref, b_ref, o_ref):
        acc = p_ref[...] + q_ref[...]
        o_ref[...] = jnp.dot(acc, w_ref[...],
                             preferred_element_type=jnp.float32) + b_ref[...]

    return pl.pallas_call(
        body,
        out_shape=jax.ShapeDtypeStruct((_G, _DOUT), jnp.float32),
    )(emb_sc, emb_tc, W, b2d)


def kernel(x, batch, y, W, b):
    bpad = jnp.pad(batch, (0, _NP - _N), constant_values=jnp.int32(1 << 30))
    bsmall = jnp.pad(bpad[::16], (0, _NS - _NP // 16),
                     constant_values=jnp.int32(1 << 30))
    emb_sc = _sc_segment_sum(x, bpad, bsmall)
    emb_tc = _tc_onehot(x, bpad.reshape(_NP // _TBLK, 1, _TBLK))
    pred = _tc_head(emb_sc, emb_tc, W, b.reshape(1, _DOUT))
    return (pred, y)
